# baseline (speedup 1.0000x reference)
import jax
import jax.numpy as jnp
from jax.experimental import pallas as pl
from jax.experimental.pallas import tpu as pltpu

_EPS = 1e-5
_B, _C, _H, _W = 32, 8, 300, 300
_SCALES = (5, 10, 15)
_TS = 32
_NS = 9
_TAIL = _H - _NS * _TS
_ROWS = 312


def _mm(a, b):
    return jax.lax.dot_general(a, b, (((1,), (0,)), ((), ())),
                               preferred_element_type=jnp.float32)


def _pool_mats(k):
    m = _H // k
    r = jax.lax.broadcasted_iota(jnp.int32, (_H, m), 0) // k
    c = jax.lax.broadcasted_iota(jnp.int32, (_H, m), 1)
    a = (r == c).astype(jnp.float32) * (1.0 / k)
    rt = jax.lax.broadcasted_iota(jnp.int32, (m, _H), 0)
    ct = jax.lax.broadcasted_iota(jnp.int32, (m, _H), 1) // k
    at = (rt == ct).astype(jnp.float32) * (1.0 / k)
    ut = (rt == ct).astype(jnp.float32)
    return a, at, ut


def _up_slab(p, k, ut, r0, ts):
    m = _H // k
    rows = jax.lax.broadcasted_iota(jnp.int32, (ts + 2, m), 0) + (r0 - 1)
    cols = jax.lax.broadcasted_iota(jnp.int32, (ts + 2, m), 1)
    udyn = (jnp.floor_divide(rows, k) == cols).astype(jnp.float32)
    uh = [_mm(udyn, p[c]) for c in range(_C)]
    uh2 = jnp.stack(uh, axis=0).reshape(_C * (ts + 2), m)
    up = _mm(uh2, ut).reshape(_C, ts + 2, _W)
    return jnp.pad(up, ((0, 0), (0, 0), (1, 1)))


def _ria_kernel(x_ref, wx_ref, wu5_ref, wu10_ref, wu15_ref, wg_ref,
                y_ref, s1_ref, s2_ref, xs_ref, cp_ref):
    x = x_ref[0]
    xs_ref[:, 0:8, :] = jnp.zeros((_C, 8, _W + 2), jnp.float32)
    xs_ref[:, _H + 8:_ROWS, :] = jnp.zeros((_C, 4, _W + 2), jnp.float32)
    xs_ref[:, 8:_H + 8, 0:1] = jnp.zeros((_C, _H, 1), jnp.float32)
    xs_ref[:, 8:_H + 8, _W + 1:_W + 2] = jnp.zeros((_C, _H, 1), jnp.float32)
    xs_ref[:, 8:_H + 8, 1:_W + 1] = x
    cp_ref[:, 0:8, :] = jnp.zeros((24, 8, _W + 2), jnp.float32)
    cp_ref[:, _H + 8:_ROWS, :] = jnp.zeros((24, 4, _W + 2), jnp.float32)
    cp_ref[:, 8:_H + 8, 0:1] = jnp.zeros((24, _H, 1), jnp.float32)
    cp_ref[:, 8:_H + 8, _W + 1:_W + 2] = jnp.zeros((24, _H, 1), jnp.float32)

    x2 = x.reshape(_C * _H, _W)
    pools, uts = [], []
    for k in _SCALES:
        a, at, ut = _pool_mats(k)
        t1 = _mm(x2, a).reshape(_C, _H, _H // k)
        pools.append(jnp.stack([_mm(at, t1[c]) for c in range(_C)], axis=0))
        uts.append(ut)

    wx = wx_ref[...]
    wus = (wu5_ref[...], wu10_ref[...], wu15_ref[...])

    def conv1_strip(r0, xsub, ts):
        ones = jnp.ones((1, ts, _W), jnp.float32)
        px = jnp.concatenate(
            [xsub[:, dh:dh + ts, dw:dw + _W]
             for dh in range(3) for dw in range(3)] + [ones], axis=0)
        cx = _mm(wx, px)
        parts = []
        for i, k in enumerate(_SCALES):
            usub = _up_slab(pools[i], k, uts[i], r0, ts)
            pu = jnp.concatenate(
                [usub[:, dh:dh + ts, dw:dw + _W]
                 for dh in range(3) for dw in range(3)], axis=0)
            parts.append(cx[8 * i:8 * i + 8] + _mm(wus[i], pu))
        return jnp.concatenate(parts, axis=0)

    def body1(s, carry):
        r0 = s * _TS
        slab = xs_ref[:, pl.ds(r0, 48), :]
        xsub = slab[:, 7:7 + _TS + 2, :]
        cp_ref[:, pl.ds(r0 + 8, _TS), 1:_W + 1] = conv1_strip(r0, xsub, _TS)
        return carry

    jax.lax.fori_loop(0, _NS, body1, 0)
    r0t = _NS * _TS
    xsub_t = xs_ref[:, r0t + 7:r0t + 7 + _TAIL + 2, :]
    cp_ref[:, r0t + 8:r0t + 8 + _TAIL, 1:_W + 1] = conv1_strip(
        r0t, xsub_t, _TAIL)

    wg = wg_ref[...]

    def gated_strip(csub, ts):
        ones = jnp.ones((1, ts, _W), jnp.float32)
        pg = jnp.concatenate(
            [csub[:, dh:dh + ts, dw:dw + _W]
             for dh in range(3) for dw in range(3)] + [ones], axis=0)
        g = _mm(wg, pg)
        return g[:_C] * jax.nn.sigmoid(g[_C:])

    def body2(s, carry):
        s1, s2 = carry
        r0 = s * _TS
        slab = cp_ref[:, pl.ds(r0, 48), :]
        ystrip = gated_strip(slab[:, 7:7 + _TS + 2, :], _TS)
        y_ref[0, :, pl.ds(r0, _TS), :] = ystrip
        return (s1 + jnp.sum(ystrip, axis=1),
                s2 + jnp.sum(ystrip * ystrip, axis=1))

    z = jnp.zeros((_C, _W), jnp.float32)
    s1, s2 = jax.lax.fori_loop(0, _NS, body2, (z, z))
    csub_t = cp_ref[:, r0t + 7:r0t + 7 + _TAIL + 2, :]
    ytail = gated_strip(csub_t, _TAIL)
    y_ref[0, :, r0t:r0t + _TAIL, :] = ytail
    s1_ref[0] = s1 + jnp.sum(ytail, axis=1)
    s2_ref[0] = s2 + jnp.sum(ytail * ytail, axis=1)


def _affine_kernel(y_ref, a_ref, b_ref, o_ref):
    o_ref[0] = y_ref[0] * a_ref[0] + b_ref[0]


def kernel(x, w5, b5, w10, b10, w15, b15, gw, gb, mw, mb, gamma, beta):
    f32 = jnp.float32

    def wmat(w, cs):
        return jnp.transpose(w[:, cs], (0, 2, 3, 1)).reshape(8, -1)

    cx = slice(0, 8)
    cu = slice(8, 16)
    wx = jnp.concatenate([wmat(w5, cx), wmat(w10, cx), wmat(w15, cx)], axis=0)
    bx = jnp.concatenate([b5, b10, b15], axis=0)[:, None]
    wx = jnp.concatenate([wx, bx], axis=1)
    wu5, wu10, wu15 = wmat(w5, cu), wmat(w10, cu), wmat(w15, cu)
    wg = jnp.concatenate([wmat(gw, slice(0, 24)), wmat(mw, slice(0, 24))],
                         axis=0)
    bg = jnp.concatenate([gb, mb], axis=0)[:, None]
    wg = jnp.concatenate([wg, bg], axis=1)

    full = lambda shape: pl.BlockSpec(shape, lambda b: (0,) * len(shape))
    y, s1, s2 = pl.pallas_call(
        _ria_kernel,
        grid=(_B,),
        in_specs=[
            pl.BlockSpec((1, _C, _H, _W), lambda b: (b, 0, 0, 0)),
            full((24, 73)), full((8, 72)), full((8, 72)), full((8, 72)),
            full((16, 217)),
        ],
        out_specs=[
            pl.BlockSpec((1, _C, _H, _W), lambda b: (b, 0, 0, 0)),
            pl.BlockSpec((1, _C, _W), lambda b: (b, 0, 0)),
            pl.BlockSpec((1, _C, _W), lambda b: (b, 0, 0)),
        ],
        out_shape=[
            jax.ShapeDtypeStruct((_B, _C, _H, _W), f32),
            jax.ShapeDtypeStruct((_B, _C, _W), f32),
            jax.ShapeDtypeStruct((_B, _C, _W), f32),
        ],
        scratch_shapes=[
            pltpu.VMEM((_C, _ROWS, _W + 2), f32),
            pltpu.VMEM((24, _ROWS, _W + 2), f32),
        ],
        compiler_params=pltpu.CompilerParams(
            dimension_semantics=('parallel',)),
    )(x, wx, wu5, wu10, wu15, wg)

    n = _B * _H * _W
    mean = jnp.sum(s1, axis=(0, 2)) / n
    var = jnp.sum(s2, axis=(0, 2)) / n - mean * mean
    scale = gamma * jax.lax.rsqrt(var + _EPS)
    shift = beta - mean * scale

    out = pl.pallas_call(
        _affine_kernel,
        grid=(_B,),
        in_specs=[
            pl.BlockSpec((1, _C, _H, _W), lambda b: (b, 0, 0, 0)),
            pl.BlockSpec((1, _C, 1, 1), lambda b: (0, 0, 0, 0)),
            pl.BlockSpec((1, _C, 1, 1), lambda b: (0, 0, 0, 0)),
        ],
        out_specs=pl.BlockSpec((1, _C, _H, _W), lambda b: (b, 0, 0, 0)),
        out_shape=jax.ShapeDtypeStruct((_B, _C, _H, _W), f32),
        compiler_params=pltpu.CompilerParams(
            dimension_semantics=('parallel',)),
    )(y, scale.reshape(1, _C, 1, 1), shift.reshape(1, _C, 1, 1))
    return out
